# baseline (speedup 1.0000x reference)
import jax
import jax.numpy as jnp
from jax import lax
from jax.experimental import pallas as pl
from jax.experimental.pallas import tpu as pltpu, tpu_sc as plsc

B = 16384
L = 50
E = 64
PAD = 64
NC = 2
NS = 16
NW = NC * NS
BPW = B // NW
CB = 8
NCH = BPW // CB
LANES = 16
NG = PAD // LANES


def _sc_body(center_hbm, ctx_hbm, v_hbm, u_hbm, out_hbm,
             cidx_v, ctx_v, ub0, ub1, vb0, vb1, ob, sem0, sem1):
    cid = lax.axis_index("c")
    sid = lax.axis_index("s")
    wid = sid * NC + cid
    base = wid * BPW

    pltpu.sync_copy(center_hbm.at[pl.ds(base, BPW)], cidx_v)
    pltpu.sync_copy(ctx_hbm.at[pl.ds(base, BPW)], ctx_v)

    def issue(c, ub, vb, sem):
        hs = [pltpu.async_copy(v_hbm.at[cidx_v.at[pl.ds(c * CB, CB)]], vb,
                               sem)]
        for i in range(CB):
            hs.append(pltpu.async_copy(u_hbm.at[ctx_v.at[c * CB + i]],
                                       ub.at[pl.ds(i * PAD, L)], sem))
        return hs

    iota = lax.iota(jnp.int32, LANES)
    rowg = [iota + g * LANES for g in range(NG)]

    def chunk(c, ub, vb, sem):
        for h in issue(c, ub, vb, sem):
            h.wait()

        for i in range(CB):
            rows = [rowg[g] + i * PAD for g in range(NG)]
            ifull = jnp.full((LANES,), i, jnp.int32)

            def ebody(e, accs, rows=rows, ifull=ifull):
                col = jnp.full((LANES,), e, jnp.int32)
                s = plsc.load_gather(vb, [ifull, col])
                return tuple(accs[g] + plsc.load_gather(ub, [rows[g], col]) * s
                             for g in range(NG))

            accs = lax.fori_loop(
                0, E, ebody,
                tuple(jnp.zeros((LANES,), jnp.float32) for _ in range(NG)))
            for g in range(NG):
                ob[i, pl.ds(g * LANES, LANES)] = accs[g]

        pltpu.sync_copy(ob, out_hbm.at[pl.ds(base + c * CB, CB)])

    def outer(c2, carry):
        chunk(c2 * 2, ub0, vb0, sem0)
        chunk(c2 * 2 + 1, ub1, vb1, sem1)
        return carry

    lax.fori_loop(0, NCH // 2, outer, 0)


def kernel(center, context_negative, v_table, u_table):
    mesh = plsc.VectorSubcoreMesh(core_axis_name="c", subcore_axis_name="s")
    padded = pl.kernel(
        _sc_body,
        out_type=jax.ShapeDtypeStruct((B, PAD), jnp.float32),
        mesh=mesh,
        compiler_params=pltpu.CompilerParams(needs_layout_passes=False,
                                             use_tc_tiling_on_sc=False),
        scratch_types=[
            pltpu.VMEM((BPW,), jnp.int32),
            pltpu.VMEM((BPW, L), jnp.int32),
            pltpu.VMEM((CB * PAD, E), jnp.float32),
            pltpu.VMEM((CB * PAD, E), jnp.float32),
            pltpu.VMEM((CB, E), jnp.float32),
            pltpu.VMEM((CB, E), jnp.float32),
            pltpu.VMEM((CB, PAD), jnp.float32),
            pltpu.SemaphoreType.DMA,
            pltpu.SemaphoreType.DMA,
        ],
    )(center.reshape(B), context_negative, v_table, u_table)
    return padded[:, :L].reshape(B, 1, L)


# speedup vs baseline: 1.8040x; 1.8040x over previous
import jax
import jax.numpy as jnp
from jax import lax
from jax.experimental import pallas as pl
from jax.experimental.pallas import tpu as pltpu, tpu_sc as plsc

B = 16384
L = 50
E = 64
PAD = 64
NC = 2
NS = 16
NW = NC * NS
BPW = B // NW
CB = 8
NCH = BPW // CB
LANES = 16
NG = PAD // LANES


def _sc_body(center_hbm, ctx_hbm, v_hbm, u_hbm, out_hbm,
             cidx_v, ctx_v, ub0, ub1, vb0, vb1, ob, sem0, sem1):
    cid = lax.axis_index("c")
    sid = lax.axis_index("s")
    wid = sid * NC + cid
    base = wid * BPW

    pltpu.sync_copy(center_hbm.at[pl.ds(base, BPW)], cidx_v)
    pltpu.sync_copy(ctx_hbm.at[pl.ds(base, BPW)], ctx_v)

    def issue(c, ub, vb, sem):
        hs = [pltpu.async_copy(v_hbm.at[cidx_v.at[pl.ds(c * CB, CB)]], vb,
                               sem)]
        for i in range(CB):
            hs.append(pltpu.async_copy(u_hbm.at[ctx_v.at[c * CB + i]],
                                       ub.at[pl.ds(i * PAD, L)], sem))
        return hs

    iota = lax.iota(jnp.int32, LANES)
    rowg = [iota + g * LANES for g in range(NG)]

    def chunk(c, ub, vb, sem):
        for h in issue(c, ub, vb, sem):
            h.wait()

        for i in range(CB):
            rows = [rowg[g] + i * PAD for g in range(NG)]
            ifull = jnp.full((LANES,), i, jnp.int32)

            def ebody(t, accs, rows=rows, ifull=ifull):
                col = (iota + t) & (E - 1)
                s = plsc.load_gather(vb, [ifull, col])
                return tuple(accs[g] + plsc.load_gather(ub, [rows[g], col]) * s
                             for g in range(NG))

            accs = lax.fori_loop(
                0, E, ebody,
                tuple(jnp.zeros((LANES,), jnp.float32) for _ in range(NG)))
            for g in range(NG):
                ob[i, pl.ds(g * LANES, LANES)] = accs[g]

        pltpu.sync_copy(ob, out_hbm.at[pl.ds(base + c * CB, CB)])

    def outer(c2, carry):
        chunk(c2 * 2, ub0, vb0, sem0)
        chunk(c2 * 2 + 1, ub1, vb1, sem1)
        return carry

    lax.fori_loop(0, NCH // 2, outer, 0)


def kernel(center, context_negative, v_table, u_table):
    mesh = plsc.VectorSubcoreMesh(core_axis_name="c", subcore_axis_name="s")
    padded = pl.kernel(
        _sc_body,
        out_type=jax.ShapeDtypeStruct((B, PAD), jnp.float32),
        mesh=mesh,
        compiler_params=pltpu.CompilerParams(needs_layout_passes=False,
                                             use_tc_tiling_on_sc=False),
        scratch_types=[
            pltpu.VMEM((BPW,), jnp.int32),
            pltpu.VMEM((BPW, L), jnp.int32),
            pltpu.VMEM((CB * PAD, E), jnp.float32),
            pltpu.VMEM((CB * PAD, E), jnp.float32),
            pltpu.VMEM((CB, E), jnp.float32),
            pltpu.VMEM((CB, E), jnp.float32),
            pltpu.VMEM((CB, PAD), jnp.float32),
            pltpu.SemaphoreType.DMA,
            pltpu.SemaphoreType.DMA,
        ],
    )(center.reshape(B), context_negative, v_table, u_table)
    return padded[:, :L].reshape(B, 1, L)
